# Initial kernel scaffold; baseline (speedup 1.0000x reference)
import jax
import jax.numpy as jnp
from jax.experimental import pallas as pl


def kernel(cam_type, camera, points, environment):
    raise NotImplementedError("write your pallas kernel here")


# speedup vs baseline: 1.3048x; 1.3048x over previous
import jax
import jax.numpy as jnp
from jax import lax
from jax.experimental import pallas as pl
from jax.experimental.pallas import tpu as pltpu
from jax.experimental.pallas import tpu_sc as plsc

H = 1024
W = 1024
NPIX = H * W
ZROWS = 172
ZSIZE = ZROWS * W
NLAUNCH = 3
NDUMP = 8
BATCH = 128
TILE_FB = ZSIZE // 16
CH = TILE_FB // 8
NCH = TILE_FB // CH
CHE = TILE_FB // 2


def _splat_body(cam_ref, zinfo_ref, pts_ref, zeros_ref,
                oa0, oa1, oa2, oa3, ob0, ob1, ob2, ob3,
                cam_v, zi_v, in_v, idx_v, v0_v, v1_v, v2_v, v3_v, buf,
                f0, f1, f2, f3):
    c = lax.axis_index("c")
    s = lax.axis_index("s")

    pltpu.sync_copy(cam_ref, cam_v)
    pltpu.sync_copy(zinfo_ref, zi_v)
    pltpu.sync_copy(zeros_ref, buf)
    planes = (f0, f1, f2, f3)
    for f in planes:
        for i in range(2):
            pltpu.sync_copy(buf, f.at[pl.ds(s * TILE_FB + i * CHE, CHE)])
    plsc.subcore_barrier()

    cam_lo = cam_v[pl.ds(0, 16)]
    cam_hi = cam_v[pl.ds(16, 16)]
    m = [cam_lo[i] for i in range(16)]
    persp = jnp.full((16,), cam_hi[0], jnp.float32)
    zi = zi_v[pl.ds(0, 16)]
    base_c = jnp.where(c == 0, zi[0], zi[1])
    iota = lax.iota(jnp.int32, 16)
    iota6 = iota * 6
    dump = ZSIZE + (iota & 7)
    one = jnp.float32(1.0)
    umax = jnp.float32(1.0 - 1e-6)

    nbatch = pts_ref.shape[0] // (BATCH * 6)
    nb = (nbatch - s + 15) // 16
    vbufs = (v0_v, v1_v, v2_v, v3_v)

    def batch(ib, carry):
        b = s + ib * 16
        pltpu.sync_copy(pts_ref.at[pl.ds(b * BATCH * 6, BATCH * 6)], in_v)
        for g in range(8):
            sl = pl.ds(g * 16, 16)
            x = plsc.load_gather(in_v, [iota6 + (g * 96 + 0)])
            y = plsc.load_gather(in_v, [iota6 + (g * 96 + 1)])
            z = plsc.load_gather(in_v, [iota6 + (g * 96 + 2)])
            r = plsc.load_gather(in_v, [iota6 + (g * 96 + 3)])
            g2 = plsc.load_gather(in_v, [iota6 + (g * 96 + 4)])
            b2 = plsc.load_gather(in_v, [iota6 + (g * 96 + 5)])
            p0 = m[0] * x + m[1] * y + m[2] * z + m[3]
            p1 = m[4] * x + m[5] * y + m[6] * z + m[7]
            p2 = m[8] * x + m[9] * y + m[10] * z + m[11]
            p3 = m[12] * x + m[13] * y + m[14] * z + m[15]
            dc = jnp.where(jnp.abs(p3) < 1e-4, jnp.float32(1e-4), p3)
            den = jnp.where(persp > 0.5, dc, one)
            u = p0 / den * 0.5 + 0.5
            v = p1 / den * 0.5 + 0.5
            u = jnp.minimum(jnp.maximum(u, 0.0), umax)
            v = jnp.minimum(jnp.maximum(v, 0.0), umax)
            px = (u * jnp.float32(W)).astype(jnp.int32)
            py = (v * jnp.float32(H)).astype(jnp.int32)
            flat = py * W + px
            w = one / (one + jnp.exp(p2))
            loc = flat - base_c
            inz = (loc >= 0) & (loc < ZSIZE)
            idx_v[sl] = jnp.where(inz, loc, dump)
            v0_v[sl] = w * r
            v1_v[sl] = w * g2
            v2_v[sl] = w * b2
            v3_v[sl] = w
        for ch in range(4):
            pltpu.sync_copy(vbufs[ch], planes[ch].at[idx_v], add=True)
        return carry

    lax.fori_loop(0, nb, batch, 0)
    plsc.subcore_barrier()

    outs_a = (oa0, oa1, oa2, oa3)
    outs_b = (ob0, ob1, ob2, ob3)
    for ch in range(4):
        for i in range(2):
            sl = pl.ds(s * TILE_FB + i * CHE, CHE)
            pltpu.sync_copy(planes[ch].at[sl], buf)

            @pl.when(c == 0)
            def _():
                pltpu.sync_copy(buf, outs_a[ch].at[sl])

            @pl.when(c == 1)
            def _():
                pltpu.sync_copy(buf, outs_b[ch].at[sl])


def _composite_body(w_ref, ar, ag, ab, er, eg, eb, pr, pg, pb):
    t = jnp.exp(-w_ref[...])
    pr[...] = er[...] * t + ar[...]
    pg[...] = eg[...] * t + ag[...]
    pb[...] = eb[...] * t + ab[...]


def kernel(cam_type, camera, points, environment):
    camera = camera.astype(jnp.float32)
    flag = jnp.where(jnp.asarray(cam_type) == 0, 1.0, 0.0).astype(jnp.float32)
    camf = jnp.concatenate(
        [camera.reshape(16), flag.reshape(1), jnp.zeros(15, jnp.float32)])
    zeros = jnp.zeros((CHE,), jnp.float32)
    mesh = plsc.VectorSubcoreMesh(
        core_axis_name="c", subcore_axis_name="s",
        num_cores=2, num_subcores=16)
    sc_params = pltpu.CompilerParams(
        needs_layout_passes=False, use_tc_tiling_on_sc=False)

    splat = pl.kernel(
        _splat_body,
        out_type=[jax.ShapeDtypeStruct((ZSIZE + NDUMP,), jnp.float32)] * 8,
        mesh=mesh,
        scratch_types=[
            pltpu.VMEM((32,), jnp.float32),
            pltpu.VMEM((16,), jnp.int32),
            pltpu.VMEM((BATCH * 6,), jnp.float32),
            pltpu.VMEM((BATCH,), jnp.int32),
            pltpu.VMEM((BATCH,), jnp.float32),
            pltpu.VMEM((BATCH,), jnp.float32),
            pltpu.VMEM((BATCH,), jnp.float32),
            pltpu.VMEM((BATCH,), jnp.float32),
            pltpu.VMEM((CHE,), jnp.float32),
            pltpu.VMEM_SHARED((ZSIZE + NDUMP,), jnp.float32),
            pltpu.VMEM_SHARED((ZSIZE + NDUMP,), jnp.float32),
            pltpu.VMEM_SHARED((ZSIZE + NDUMP,), jnp.float32),
            pltpu.VMEM_SHARED((ZSIZE + NDUMP,), jnp.float32),
        ],
        compiler_params=sc_params,
    )

    pieces = [[] for _ in range(4)]
    for k in range(NLAUNCH):
        base_a = 2 * k * ZSIZE
        zinfo = jnp.array([base_a, base_a + ZSIZE] + [0] * 14, jnp.int32)
        outs = splat(camf, zinfo, points.reshape(-1), zeros)
        for ch in range(4):
            pieces[ch].append(outs[ch][:ZSIZE])
            pieces[ch].append(outs[4 + ch][:ZSIZE])
    tail = NPIX - (2 * NLAUNCH - 1) * ZSIZE
    for ch in range(4):
        pieces[ch][-1] = pieces[ch][-1][:tail]
    chans = [jnp.concatenate(pieces[ch], axis=0) for ch in range(4)]
    wsum = chans[3].reshape(H, W)
    accc = [chans[i].reshape(H, W) for i in range(3)]
    envc = [environment[:, :, i].astype(jnp.float32) for i in range(3)]

    blk = pl.BlockSpec((128, W), lambda i: (i, 0))
    prgb = pl.pallas_call(
        _composite_body,
        grid=(H // 128,),
        in_specs=[blk] * 7,
        out_specs=[blk] * 3,
        out_shape=[jax.ShapeDtypeStruct((H, W), jnp.float32)] * 3,
    )(wsum, *accc, *envc)
    return jnp.stack(prgb, axis=-1)
